# baseline (speedup 1.0000x reference)
import jax
import jax.numpy as jnp
from jax.experimental import pallas as pl
from jax.experimental.pallas import tpu as pltpu

_CHUNK_ROWS = 8
_N_CHUNKS = 25
_SPLIT = 12


def _wide_kernel(x_hbm, w_hbm, b_ref, o_ref, x_vmem, w_vmem, sem_x, sem_w):
    w_cp = pltpu.make_async_copy(w_hbm, w_vmem, sem_w)
    x_cps = [
        pltpu.make_async_copy(
            x_hbm.at[pl.ds(i * _CHUNK_ROWS, _CHUNK_ROWS), :],
            x_vmem.at[pl.ds(i * _CHUNK_ROWS, _CHUNK_ROWS), :],
            sem_x.at[i],
        )
        for i in range(_N_CHUNKS)
    ]
    w_cp.start()
    for cp in x_cps:
        cp.start()
    w_cp.wait()
    w = w_vmem[...]
    for i in range(_SPLIT):
        x_cps[i].wait()
    r0 = _SPLIT * _CHUNK_ROWS
    acc = jnp.dot(
        w[:, :r0], x_vmem[:r0, :], preferred_element_type=jnp.float32
    )
    for i in range(_SPLIT, _N_CHUNKS):
        x_cps[i].wait()
    acc = acc + jnp.dot(
        w[:, r0:], x_vmem[r0:, :], preferred_element_type=jnp.float32
    )
    o_ref[...] = acc + b_ref[...][:, None]


def kernel(feat, server_model, len_seq, mask, manfeat, emb1_w, emb2_w, emb3_w,
           emb4_w, emb5_w, k_w, o_w, cls_w, cls_b, wide_w, wide_b):
    b, k = manfeat.shape
    n = wide_w.shape[1]
    xt = manfeat.T
    xt = pltpu.with_memory_space_constraint(xt, pltpu.MemorySpace.HBM)
    wt = wide_w.T
    wt = pltpu.with_memory_space_constraint(wt, pltpu.MemorySpace.HBM)
    out_t = pl.pallas_call(
        _wide_kernel,
        in_specs=[
            pl.BlockSpec(memory_space=pltpu.MemorySpace.HBM),
            pl.BlockSpec(memory_space=pltpu.MemorySpace.HBM),
            pl.BlockSpec(memory_space=pltpu.MemorySpace.VMEM),
        ],
        out_specs=pl.BlockSpec(memory_space=pltpu.MemorySpace.VMEM),
        out_shape=jax.ShapeDtypeStruct((n, b), jnp.float32),
        scratch_shapes=[
            pltpu.VMEM((k, b), jnp.float32),
            pltpu.VMEM((n, k), jnp.float32),
            pltpu.SemaphoreType.DMA((_N_CHUNKS,)),
            pltpu.SemaphoreType.DMA,
        ],
        compiler_params=pltpu.CompilerParams(
            disable_bounds_checks=True,
            disable_semaphore_checks=True,
        ),
    )(xt, wt, wide_b)
    return out_t.T


# speedup vs baseline: 1.2185x; 1.2185x over previous
import jax
import jax.numpy as jnp
from jax.experimental import pallas as pl
from jax.experimental.pallas import tpu as pltpu


def _wide_kernel(w_ref, x_ref, b_ref, o_ref):
    o_ref[...] = (
        jnp.dot(w_ref[...], x_ref[...], preferred_element_type=jnp.float32)
        + b_ref[...][:, None]
    )


def kernel(feat, server_model, len_seq, mask, manfeat, emb1_w, emb2_w, emb3_w,
           emb4_w, emb5_w, k_w, o_w, cls_w, cls_b, wide_w, wide_b):
    b, k = manfeat.shape
    n = wide_w.shape[1]
    xt = manfeat.T
    wt = wide_w.T
    out_t = pl.pallas_call(
        _wide_kernel,
        out_shape=jax.ShapeDtypeStruct((n, b), jnp.float32),
        compiler_params=pltpu.CompilerParams(
            disable_bounds_checks=True,
            disable_semaphore_checks=True,
        ),
    )(wt, xt, wide_b)
    return out_t.T
